# Initial kernel scaffold; baseline (speedup 1.0000x reference)
import jax
import jax.numpy as jnp
from jax.experimental import pallas as pl


def kernel(sparse_feat, dense_feat, W_sparse, w_dense, b):
    raise NotImplementedError("write your pallas kernel here")


# speedup vs baseline: 1.2488x; 1.2488x over previous
import functools

import jax
import jax.numpy as jnp
from jax import lax
from jax.experimental import pallas as pl
from jax.experimental.pallas import tpu as pltpu
from jax.experimental.pallas import tpu_sc as plsc

_B, _F, _V, _D = 16384, 26, 100000, 13
_NC, _NS, _L = 2, 16, 16
_NW = _NC * _NS
_CB = _B // _NW
_CH = 128
_NCH = _CB // _CH


def _body(sf_hbm, dn_hbm, tbl_hbm, wb_hbm, out_hbm,
          idx_v, g_v, dn_v, wb_v, out_v, sem):
  wid = lax.axis_index("s") * _NC + lax.axis_index("c")
  base = wid * _CB

  pltpu.sync_copy(sf_hbm.at[:, pl.ds(wid * _NCH, _NCH), :], idx_v)
  pltpu.sync_copy(dn_hbm.at[:, pl.ds(base, _CB)], dn_v)
  pltpu.sync_copy(wb_hbm, wb_v)

  def add_off(c, carry):
    for f in range(_F):
      off = jnp.int32(f * _V)
      for l in range(_CH // _L):
        s = pl.ds(l * _L, _L)
        idx_v[f, c, s] = idx_v[f, c, s] + off
    return carry
  lax.fori_loop(0, _NCH, add_off, 0)

  copies = []
  for f in range(_F):
    for c in range(_NCH):
      copies.append(
          pltpu.async_copy(tbl_hbm.at[idx_v.at[f, c]], g_v.at[f, c], sem))
  for cp in copies:
    cp.wait()

  wvec = wb_v[pl.ds(0, _L)]
  bias = wvec[_D]

  def reduce_c(c, carry):
    for l in range(_CH // _L):
      s = pl.ds(l * _L, _L)
      acc = jnp.full((_L,), bias, jnp.float32)
      for f in range(_F):
        acc = acc + g_v[f, c, s]
      s2 = pl.ds(c * _CH + l * _L, _L)
      for d in range(_D):
        acc = acc + wvec[d] * dn_v[d, s2]
      out_v[s2] = acc
    return carry
  lax.fori_loop(0, _NCH, reduce_c, 0)

  pltpu.sync_copy(out_v, out_hbm.at[pl.ds(base, _CB)])


@jax.jit
def _run(sf_r, dn_t, tbl, wb):
  mesh = plsc.VectorSubcoreMesh(core_axis_name="c", subcore_axis_name="s")
  return pl.kernel(
      _body,
      out_type=jax.ShapeDtypeStruct((_B,), jnp.float32),
      mesh=mesh,
      scratch_types=[
          pltpu.VMEM((_F, _NCH, _CH), jnp.int32),
          pltpu.VMEM((_F, _NCH, _CH), jnp.float32),
          pltpu.VMEM((_D, _CB), jnp.float32),
          pltpu.VMEM((_L,), jnp.float32),
          pltpu.VMEM((_CB,), jnp.float32),
          pltpu.SemaphoreType.DMA,
      ],
  )(sf_r, dn_t, tbl, wb)


def kernel(sparse_feat, dense_feat, W_sparse, w_dense, b):
  sf_r = sparse_feat.astype(jnp.int32).T.reshape(_F, _NW * _NCH, _CH)
  dn_t = dense_feat.astype(jnp.float32).T
  tbl = W_sparse.reshape(_F * _V)
  wb = jnp.concatenate(
      [w_dense, b, jnp.zeros((_L - _D - 1,), jnp.float32)])
  return _run(sf_r, dn_t, tbl, wb)
